# baseline (speedup 1.0000x reference)
import functools

import jax
import jax.numpy as jnp
from jax import lax
from jax.experimental import pallas as pl
from jax.experimental.pallas import tpu as pltpu
from jax.experimental.pallas import tpu_sc as plsc

B, S, D = 1024, 50, 32
MAXC = 8
N = B * S
NW = 32
BCH = 128
BT = B // BCH
NQ = S * BT
NITER = (NQ + NW - 1) // NW
CHUNK_LOOK = BCH * MAXC


def _gather_sum_body(idx_hbm, table_hbm, out_hbm,
                     idx_a, idx_b, rows_a, rows_b, out_a, out_b,
                     sem_a, sem_b, sem_o):
    wid = lax.axis_index("s") * 2 + lax.axis_index("c")
    bufs = [(idx_a, rows_a, out_a, sem_a), (idx_b, rows_b, out_b, sem_b)]

    def chunk_q(i):
        q = i * NW + wid
        return q - (q // NQ) * NQ

    def stage(i):
        idx_v, rows_v, _, sem = bufs[i % 2]
        q = chunk_q(i)
        pltpu.sync_copy(idx_hbm.at[q // BT, q % BT], idx_v)
        return [
            pltpu.async_copy(table_hbm.at[idx_v.at[k]],
                             rows_v.at[pl.ds(k * BCH, BCH)], sem)
            for k in range(MAXC)
        ]

    def finish(i, descs):
        _, rows_v, out_v, _ = bufs[i % 2]
        for dsc in descs:
            dsc.wait()

        def body(j, carry):
            a0 = rows_v[j, 0:16]
            a1 = rows_v[j, 16:32]
            for k in range(1, MAXC):
                a0 = a0 + rows_v[k * BCH + j, 0:16]
                a1 = a1 + rows_v[k * BCH + j, 16:32]
            col = (j % 4) * D
            out_v[j // 4, pl.ds(col, 16)] = a0
            out_v[j // 4, pl.ds(col + 16, 16)] = a1
            return carry

        lax.fori_loop(0, BCH, body, 0)
        q = chunk_q(i)
        pltpu.async_copy(
            out_v, out_hbm.at[q // BT, pl.ds((q % BT) * (BCH // 4), BCH // 4)],
            sem_o).wait()

    descs = stage(0)
    for i in range(NITER):
        nxt = stage(i + 1) if i + 1 < NITER else None
        finish(i, descs)
        descs = nxt


@functools.cache
def _gather_sum():
    return pl.kernel(
        _gather_sum_body,
        out_type=jax.ShapeDtypeStruct((S, B // 4, 4 * D), jnp.float32),
        mesh=plsc.VectorSubcoreMesh(core_axis_name="c", subcore_axis_name="s"),
        compiler_params=pltpu.CompilerParams(use_tc_tiling_on_sc=False),
        scratch_types=[
            pltpu.VMEM((MAXC, BCH), jnp.int32),
            pltpu.VMEM((MAXC, BCH), jnp.int32),
            pltpu.VMEM((CHUNK_LOOK, D), jnp.float32),
            pltpu.VMEM((CHUNK_LOOK, D), jnp.float32),
            pltpu.VMEM((BCH // 4, 4 * D), jnp.float32),
            pltpu.VMEM((BCH // 4, 4 * D), jnp.float32),
            pltpu.SemaphoreType.DMA,
            pltpu.SemaphoreType.DMA,
            pltpu.SemaphoreType.DMA,
        ],
    )


def _epilogue_body(c_ref, resp_ref, cn_ref, ed_ref, rt_ref, p_ref, pm_ref,
                   o1, o2, o3, o4, o5, o6):
    x = c_ref[0]
    ct = jnp.transpose(x.reshape(B // 4, 4, D), (2, 0, 1)).reshape(D, B)
    resp = resp_ref[0]
    cn = cn_ref[0]
    ed = ed_ref[0]
    rt = rt_ref[...]
    pcol = p_ref[0]
    pm = pm_ref[...]
    cw = ct / jnp.where(cn == 0, 1, cn).astype(jnp.float32)
    bp = 1.0 - ed
    ep1 = bp * pm[:, 0:1] + pm[:, 1:2]
    ep = jnp.sum(ep1 * pm[:, 2:3], axis=0, keepdims=True) + pm[0:1, 3:4]
    e = cw + ep
    is1 = resp == 1
    r = jnp.where(is1, rt[:, 1:2], rt[:, 0:1])
    top = jnp.where(is1, e, r)
    bot = jnp.where(is1, r, e)
    o1[0, 0:D, :] = top + pcol[0:D]
    o1[0, D:2 * D, :] = bot + pcol[D:2 * D]
    o2[0, 0:D, :] = jnp.broadcast_to(rt[:, 2:3], e.shape)
    o2[0, D:2 * D, :] = e
    o3[0] = ep
    o4[0] = cw
    o5[0, 0:D, :] = top
    o5[0, D:2 * D, :] = bot
    o6[0, 0:D, :] = bot + pcol[0:D]
    o6[0, D:2 * D, :] = top + pcol[D:2 * D]


_epilogue = pl.pallas_call(
    _epilogue_body,
    grid=(S,),
    in_specs=[
        pl.BlockSpec((1, B // 4, 4 * D), lambda g: (g, 0, 0)),
        pl.BlockSpec((1, 1, B), lambda g: (g, 0, 0)),
        pl.BlockSpec((1, 1, B), lambda g: (g, 0, 0)),
        pl.BlockSpec((1, 1, B), lambda g: (g, 0, 0)),
        pl.BlockSpec((D, 4), lambda g: (0, 0)),
        pl.BlockSpec((1, 2 * D, 1), lambda g: (g, 0, 0)),
        pl.BlockSpec((D, 4), lambda g: (0, 0)),
    ],
    out_specs=[
        pl.BlockSpec((1, 2 * D, B), lambda g: (g, 0, 0)),
        pl.BlockSpec((1, 2 * D, B), lambda g: (g, 0, 0)),
        pl.BlockSpec((1, 1, B), lambda g: (g, 0, 0)),
        pl.BlockSpec((1, D, B), lambda g: (g, 0, 0)),
        pl.BlockSpec((1, 2 * D, B), lambda g: (g, 0, 0)),
        pl.BlockSpec((1, 2 * D, B), lambda g: (g, 0, 0)),
    ],
    out_shape=[
        jax.ShapeDtypeStruct((S, 2 * D, B), jnp.float32),
        jax.ShapeDtypeStruct((S, 2 * D, B), jnp.float32),
        jax.ShapeDtypeStruct((S, 1, B), jnp.float32),
        jax.ShapeDtypeStruct((S, D, B), jnp.float32),
        jax.ShapeDtypeStruct((S, 2 * D, B), jnp.float32),
        jax.ShapeDtypeStruct((S, 2 * D, B), jnp.float32),
    ],
)


def kernel(exercises, categories, cate_num, exe_diff, lt_s, lt_m, lt_d,
           responses, category_table, response_table, position_table,
           b_param_W, b_param_b, b_param2_W, b_param2_b):
    t128 = jnp.reshape(category_table, (1000000 * D // 128, 128))
    t128 = lax.optimization_barrier(t128)
    table_rm = jnp.reshape(t128, (1000000, D))
    idx = jnp.transpose(categories.astype(jnp.int32), (1, 2, 0))
    idx = idx.reshape(S, MAXC, BT, BCH).transpose(0, 2, 1, 3)
    c_t = _gather_sum()(idx, table_rm)
    resp = jnp.transpose(responses.astype(jnp.int32)).reshape(S, 1, B)
    cn = jnp.transpose(cate_num.astype(jnp.int32)).reshape(S, 1, B)
    ed = jnp.transpose(exe_diff.astype(jnp.float32)).reshape(S, 1, B)
    rt_t = jnp.transpose(response_table)
    p_t = position_table.reshape(S, 2 * D, 1)
    pm = jnp.stack([
        b_param_W[:, 0], b_param_b, b_param2_W[0],
        jnp.full((D,), b_param2_b[0], jnp.float32),
    ], axis=1)
    o1, o2, o3, o4, o5, o6 = _epilogue(c_t, resp, cn, ed, rt_t, p_t, pm)
    return (jnp.transpose(o1, (2, 0, 1)), jnp.transpose(o2, (2, 0, 1)),
            jnp.transpose(o3, (2, 0, 1)), jnp.transpose(o4, (2, 0, 1)),
            jnp.transpose(o5, (2, 0, 1)), jnp.transpose(o6, (2, 0, 1)))


# speedup vs baseline: 1.1091x; 1.1091x over previous
import functools

import jax
import jax.numpy as jnp
from jax import lax
from jax.experimental import pallas as pl
from jax.experimental.pallas import tpu as pltpu
from jax.experimental.pallas import tpu_sc as plsc

B, S, D = 1024, 50, 32
MAXC = 8
N = B * S
NW = 32
BCH = 128
BT = B // BCH
NQ = S * BT
NITER = (NQ + NW - 1) // NW
CHUNK_LOOK = BCH * MAXC


def _gather_sum_body(idx_hbm, table_hbm, out_hbm,
                     idx_a, idx_b, rows_a, rows_b, out_a, out_b,
                     sem_a, sem_b, sem_o):
    wid = lax.axis_index("s") * 2 + lax.axis_index("c")
    bufs = [(idx_a, rows_a, out_a, sem_a), (idx_b, rows_b, out_b, sem_b)]

    def chunk_q(i):
        q = i * NW + wid
        return q - (q // NQ) * NQ

    def stage(i):
        idx_v, rows_v, _, sem = bufs[i % 2]
        q = chunk_q(i)
        pltpu.sync_copy(idx_hbm.at[q // BT, q % BT], idx_v)
        return [
            pltpu.async_copy(table_hbm.at[idx_v.at[k]],
                             rows_v.at[pl.ds(k * BCH, BCH)], sem)
            for k in range(MAXC)
        ]

    def finish(i, descs):
        _, rows_v, out_v, _ = bufs[i % 2]
        for dsc in descs:
            dsc.wait()

        def body(j, carry):
            a0 = rows_v[j, 0:16]
            a1 = rows_v[j, 16:32]
            for k in range(1, MAXC):
                a0 = a0 + rows_v[k * BCH + j, 0:16]
                a1 = a1 + rows_v[k * BCH + j, 16:32]
            out_v[j, 0:16] = a0
            out_v[j, 16:32] = a1
            return carry

        lax.fori_loop(0, BCH, body, 0)
        q = chunk_q(i)
        pltpu.async_copy(out_v, out_hbm.at[q // BT, pl.ds((q % BT) * BCH, BCH)],
                         sem_o).wait()

    descs = stage(0)
    for i in range(NITER):
        nxt = stage(i + 1) if i + 1 < NITER else None
        finish(i, descs)
        descs = nxt


@functools.cache
def _gather_sum():
    return pl.kernel(
        _gather_sum_body,
        out_type=jax.ShapeDtypeStruct((S, B, D), jnp.float32),
        mesh=plsc.VectorSubcoreMesh(core_axis_name="c", subcore_axis_name="s"),
        compiler_params=pltpu.CompilerParams(use_tc_tiling_on_sc=False),
        scratch_types=[
            pltpu.VMEM((MAXC, BCH), jnp.int32),
            pltpu.VMEM((MAXC, BCH), jnp.int32),
            pltpu.VMEM((CHUNK_LOOK, D), jnp.float32),
            pltpu.VMEM((CHUNK_LOOK, D), jnp.float32),
            pltpu.VMEM((BCH, D), jnp.float32),
            pltpu.VMEM((BCH, D), jnp.float32),
            pltpu.SemaphoreType.DMA,
            pltpu.SemaphoreType.DMA,
            pltpu.SemaphoreType.DMA,
        ],
    )


def _epilogue_body(c_ref, resp_ref, cn_ref, ed_ref, rt_ref, p_ref, pm_ref,
                   o1, o2, o3, o4, o5, o6):
    ct = c_ref[0]
    resp = resp_ref[0]
    cn = cn_ref[0]
    ed = ed_ref[0]
    rt = rt_ref[...]
    pcol = p_ref[0]
    pm = pm_ref[...]
    cw = ct / jnp.where(cn == 0, 1, cn).astype(jnp.float32)
    bp = 1.0 - ed
    ep1 = bp * pm[:, 0:1] + pm[:, 1:2]
    ep = jnp.sum(ep1 * pm[:, 2:3], axis=0, keepdims=True) + pm[0:1, 3:4]
    e = cw + ep
    is1 = resp == 1
    r = jnp.where(is1, rt[:, 1:2], rt[:, 0:1])
    top = jnp.where(is1, e, r)
    bot = jnp.where(is1, r, e)
    o1[0, 0:D, :] = top + pcol[0:D]
    o1[0, D:2 * D, :] = bot + pcol[D:2 * D]
    o2[0, 0:D, :] = jnp.broadcast_to(rt[:, 2:3], e.shape)
    o2[0, D:2 * D, :] = e
    o3[0] = ep
    o4[0] = cw
    o5[0, 0:D, :] = top
    o5[0, D:2 * D, :] = bot
    o6[0, 0:D, :] = bot + pcol[0:D]
    o6[0, D:2 * D, :] = top + pcol[D:2 * D]


_epilogue = pl.pallas_call(
    _epilogue_body,
    grid=(S,),
    in_specs=[
        pl.BlockSpec((1, D, B), lambda g: (g, 0, 0)),
        pl.BlockSpec((1, 1, B), lambda g: (g, 0, 0)),
        pl.BlockSpec((1, 1, B), lambda g: (g, 0, 0)),
        pl.BlockSpec((1, 1, B), lambda g: (g, 0, 0)),
        pl.BlockSpec((D, 4), lambda g: (0, 0)),
        pl.BlockSpec((1, 2 * D, 1), lambda g: (g, 0, 0)),
        pl.BlockSpec((D, 4), lambda g: (0, 0)),
    ],
    out_specs=[
        pl.BlockSpec((1, 2 * D, B), lambda g: (g, 0, 0)),
        pl.BlockSpec((1, 2 * D, B), lambda g: (g, 0, 0)),
        pl.BlockSpec((1, 1, B), lambda g: (g, 0, 0)),
        pl.BlockSpec((1, D, B), lambda g: (g, 0, 0)),
        pl.BlockSpec((1, 2 * D, B), lambda g: (g, 0, 0)),
        pl.BlockSpec((1, 2 * D, B), lambda g: (g, 0, 0)),
    ],
    out_shape=[
        jax.ShapeDtypeStruct((S, 2 * D, B), jnp.float32),
        jax.ShapeDtypeStruct((S, 2 * D, B), jnp.float32),
        jax.ShapeDtypeStruct((S, 1, B), jnp.float32),
        jax.ShapeDtypeStruct((S, D, B), jnp.float32),
        jax.ShapeDtypeStruct((S, 2 * D, B), jnp.float32),
        jax.ShapeDtypeStruct((S, 2 * D, B), jnp.float32),
    ],
)


def kernel(exercises, categories, cate_num, exe_diff, lt_s, lt_m, lt_d,
           responses, category_table, response_table, position_table,
           b_param_W, b_param_b, b_param2_W, b_param2_b):
    t128 = jnp.reshape(category_table, (1000000 * D // 128, 128))
    t128 = lax.optimization_barrier(t128)
    table_rm = jnp.reshape(t128, (1000000, D))
    idx = jnp.transpose(categories.astype(jnp.int32), (1, 2, 0))
    idx = idx.reshape(S, MAXC, BT, BCH).transpose(0, 2, 1, 3)
    c = _gather_sum()(idx, table_rm)
    c_t = jnp.swapaxes(c, 1, 2)
    resp = jnp.transpose(responses.astype(jnp.int32)).reshape(S, 1, B)
    cn = jnp.transpose(cate_num.astype(jnp.int32)).reshape(S, 1, B)
    ed = jnp.transpose(exe_diff.astype(jnp.float32)).reshape(S, 1, B)
    rt_t = jnp.transpose(response_table)
    p_t = position_table.reshape(S, 2 * D, 1)
    pm = jnp.stack([
        b_param_W[:, 0], b_param_b, b_param2_W[0],
        jnp.full((D,), b_param2_b[0], jnp.float32),
    ], axis=1)
    o1, o2, o3, o4, o5, o6 = _epilogue(c_t, resp, cn, ed, rt_t, p_t, pm)
    return (jnp.transpose(o1, (2, 0, 1)), jnp.transpose(o2, (2, 0, 1)),
            jnp.transpose(o3, (2, 0, 1)), jnp.transpose(o4, (2, 0, 1)),
            jnp.transpose(o5, (2, 0, 1)), jnp.transpose(o6, (2, 0, 1)))
